# baseline (speedup 1.0000x reference)
import functools

import jax
import jax.numpy as jnp
from jax import lax
from jax.experimental import pallas as pl
from jax.experimental.pallas import tpu as pltpu
from jax.experimental.pallas import tpu_sc as plsc

EMBED_DIM = 32
LANES = 16


@functools.cache
def _build(batch, num_users, num_items):
    info = plsc.get_sparse_core_info()
    nw = info.num_cores * info.num_subcores
    bpw = batch // nw
    groups = bpw // LANES
    mesh = plsc.VectorSubcoreMesh(core_axis_name="c", subcore_axis_name="s")

    @functools.partial(
        pl.kernel,
        out_type=jax.ShapeDtypeStruct((batch,), jnp.float32),
        mesh=mesh,
        compiler_params=pltpu.CompilerParams(
            needs_layout_passes=False, use_tc_tiling_on_sc=False),
        scratch_types=[
            pltpu.VMEM((bpw,), jnp.int32),
            pltpu.VMEM((bpw,), jnp.int32),
            pltpu.VMEM((bpw, EMBED_DIM), jnp.float32),
            pltpu.VMEM((bpw, EMBED_DIM), jnp.float32),
            pltpu.VMEM((bpw,), jnp.float32),
            pltpu.VMEM((bpw,), jnp.float32),
            pltpu.VMEM((bpw,), jnp.float32),
            pltpu.VMEM((16,), jnp.float32),
            pltpu.SemaphoreType.DMA,
            pltpu.SemaphoreType.DMA,
            pltpu.SemaphoreType.DMA,
            pltpu.SemaphoreType.DMA,
        ],
    )
    def mf_kernel(user_ids, item_ids, user_emb, item_emb, user_bias,
                  item_bias, global_bias, out,
                  idx_u, idx_i, u_rows, i_rows, u_b, i_b, out_v, gb_s,
                  sem_u, sem_i, sem_ub, sem_ib):
        wid = lax.axis_index("s") * info.num_cores + lax.axis_index("c")
        base = wid * bpw

        pltpu.sync_copy(user_ids.at[pl.ds(base, bpw)], idx_u)
        pltpu.sync_copy(item_ids.at[pl.ds(base, bpw)], idx_i)

        cu = pltpu.async_copy(user_emb.at[idx_u], u_rows, sem_u)
        ci = pltpu.async_copy(item_emb.at[idx_i], i_rows, sem_i)
        cub = pltpu.async_copy(user_bias.at[idx_u], u_b, sem_ub)
        cib = pltpu.async_copy(item_bias.at[idx_i], i_b, sem_ib)
        pltpu.sync_copy(global_bias.at[pl.ds(0, 1)], gb_s.at[pl.ds(0, 1)])
        cu.wait()
        ci.wait()
        cub.wait()
        cib.wait()

        gb = gb_s[...][0]
        lanes = lax.iota(jnp.int32, LANES)

        def body(g, carry):
            rows = g * LANES + lanes
            ub = plsc.load_gather(u_b, [rows])
            ib = plsc.load_gather(i_b, [rows])
            acc = ub + ib + gb
            for d in range(EMBED_DIM):
                cols = jnp.full((LANES,), d, jnp.int32)
                uu = plsc.load_gather(u_rows, [rows, cols])
                vv = plsc.load_gather(i_rows, [rows, cols])
                acc = acc + uu * vv
            out_v[pl.ds(g * LANES, LANES)] = acc
            return carry

        lax.fori_loop(0, groups, body, 0)
        pltpu.sync_copy(out_v, out.at[pl.ds(base, bpw)])

    return mf_kernel


def kernel(user_ids, item_ids, user_embedding, item_embedding, user_bias,
           item_bias, global_bias):
    fn = _build(user_ids.shape[0], user_embedding.shape[0],
                item_embedding.shape[0])
    return fn(user_ids.astype(jnp.int32), item_ids.astype(jnp.int32),
              user_embedding, item_embedding,
              user_bias.reshape(-1), item_bias.reshape(-1),
              global_bias)
